# baseline (speedup 1.0000x reference)
import jax
import jax.numpy as jnp
from jax.experimental import pallas as pl
from jax.experimental.pallas import tpu as pltpu

_BEAM = 4
_END = 2
_NEG = -3.0e38
_BIG = 2147483647
_LANES = 128
_SLICES = 32
_TREES = 4
_SPT = _SLICES // _TREES
_CHUNK = _LANES * _SLICES
_RB = 32
_INV_PEN = 1.0


def _leaf(va, ia, vb, ib):
    c = vb > va
    return (jnp.where(c, vb, va), jnp.where(c, ib, ia),
            jnp.where(c, va, vb), jnp.where(c, ia, ib))


def _comb(a, b):
    a1, ai1, a2, ai2, ad = a
    b1, bi1, b2, bi2, bd = b
    cm = b1 > a1
    o1 = jnp.where(cm, b1, a1)
    oi1 = jnp.where(cm, bi1, ai1)
    x = jnp.where(cm, a1, b1)
    xi = jnp.where(cm, ai1, bi1)
    cy = b2 > a2
    y = jnp.where(cy, b2, a2)
    yi = jnp.where(cy, bi2, ai2)
    c2 = y > x
    o2 = jnp.where(c2, y, x)
    oi2 = jnp.where(c2, yi, xi)
    r3 = jnp.where(c2, x, y)
    d = jnp.maximum(jnp.maximum(ad, bd), r3)
    return (o1, oi1, o2, oi2, d)


def _scan_body(vocab, nchunks, lp_ref, tk_ref, val_ref, idx_ref, fb_ref,
               m_ref, i_ref, d_ref):
    c = pl.program_id(0)
    rows = lp_ref.shape[0]

    @pl.when(c == 0)
    def _init():
        m_ref[...] = jnp.full(m_ref.shape, _NEG, jnp.float32)
        i_ref[...] = jnp.zeros(i_ref.shape, jnp.int32)
        d_ref[...] = jnp.full(d_ref.shape, _NEG, jnp.float32)

    nrb = rows // _RB
    for rb in range(nrb):
        r0 = rb * _RB
        lane = jax.lax.broadcasted_iota(jnp.int32, (_RB, _LANES), 1)
        tk = tk_ref[r0:r0 + _RB, :]
        for t in range(_TREES):
            tnode = None
            for p in range(_SPT // 2):
                s0 = _SPT * t + 2 * p
                vs = []
                cs = []
                for q in range(2):
                    s = s0 + q
                    base = c * _CHUNK + s * _LANES
                    v = lp_ref[r0:r0 + _RB,
                               s * _LANES:(s + 1) * _LANES] + tk
                    v = jnp.where(lane < (vocab - base), v, _NEG)
                    if t == 0 and p == 0 and q == 0:
                        v = jnp.where((c == 0) & (lane == _END), _NEG, v)
                    vs.append(v)
                    cs.append(lane + base)
                m1, i1, m2, i2 = _leaf(vs[0], cs[0], vs[1], cs[1])
                node = (m1, i1, m2, i2, jnp.full((_RB, _LANES), _NEG,
                                                 jnp.float32))
                tnode = node if tnode is None else _comb(tnode, node)
            st = (m_ref[t, 0, r0:r0 + _RB, :], i_ref[t, 0, r0:r0 + _RB, :],
                  m_ref[t, 1, r0:r0 + _RB, :], i_ref[t, 1, r0:r0 + _RB, :],
                  d_ref[t, r0:r0 + _RB, :])
            n1, ni1, n2, ni2, nd = _comb(st, tnode)
            m_ref[t, 0, r0:r0 + _RB, :] = n1
            i_ref[t, 0, r0:r0 + _RB, :] = ni1
            m_ref[t, 1, r0:r0 + _RB, :] = n2
            i_ref[t, 1, r0:r0 + _RB, :] = ni2
            d_ref[t, r0:r0 + _RB, :] = nd

    @pl.when(c == nchunks - 1)
    def _extract():
        Ms = [m_ref[t, k] for t in range(_TREES) for k in range(2)]
        Xs = [i_ref[t, k] for t in range(_TREES) for k in range(2)]
        vs, xs = [], []
        for r in range(_BEAM):
            g = Ms[0]
            for j in range(1, 8):
                g = jnp.maximum(g, Ms[j])
            m = jnp.max(g, axis=1, keepdims=True)
            am = jnp.where(Ms[0] == m, Xs[0], _BIG)
            for j in range(1, 8):
                am = jnp.minimum(am, jnp.where(Ms[j] == m, Xs[j], _BIG))
            am = jnp.min(am, axis=1, keepdims=True)
            vs.append(m)
            xs.append(am)
            if r < _BEAM - 1:
                Ms = [jnp.where(Xs[j] == am, _NEG, Ms[j]) for j in range(8)]
        val_ref[...] = jnp.concatenate(vs, axis=1)
        idx_ref[...] = jnp.concatenate(xs, axis=1)
        tau = vs[_BEAM - 1]
        fb = (d_ref[0] >= tau)
        for t in range(1, _TREES):
            fb = fb | (d_ref[t] >= tau)
        fb_ref[...] = jnp.max(fb.astype(jnp.int32), axis=1, keepdims=True)


def _slow_body(vocab, lp_ref, tk_ref, val_ref, idx_ref):
    i = pl.program_id(0)
    rows, chunk = lp_ref.shape

    @pl.when(i == 0)
    def _init():
        val_ref[...] = jnp.full(val_ref.shape, _NEG, jnp.float32)
        idx_ref[...] = jnp.zeros(idx_ref.shape, jnp.int32)

    x = lp_ref[...] + tk_ref[...]
    col = jax.lax.broadcasted_iota(jnp.int32, (rows, chunk), 1) + i * chunk
    ok = (col < vocab) & (col != _END)
    x = jnp.where(ok, x, _NEG)
    vals = [val_ref[...]]
    idxs = [idx_ref[...]]
    for j in range(_BEAM):
        m = jnp.max(x, axis=1, keepdims=True)
        am = jnp.min(jnp.where(x == m, col, _BIG), axis=1, keepdims=True)
        vals.append(m)
        idxs.append(am)
        if j < _BEAM - 1:
            x = jnp.where(col == am, _NEG, x)
    allv = jnp.concatenate(vals, axis=1)
    alli = jnp.concatenate(idxs, axis=1)
    nv, ni = [], []
    for j in range(_BEAM):
        m = jnp.max(allv, axis=1, keepdims=True)
        am = jnp.min(jnp.where(allv == m, alli, _BIG), axis=1, keepdims=True)
        nv.append(m)
        ni.append(am)
        if j < _BEAM - 1:
            allv = jnp.where(alli == am, _NEG, allv)
    val_ref[...] = jnp.concatenate(nv, axis=1)
    idx_ref[...] = jnp.concatenate(ni, axis=1)


def _merge_body(vocab, cur_len,
                v_ref, t_ref, gb_ref, off_ref,
                lp_ref, sc_ref, tok_ref, rows_ref, gb_out_ref, fin_ref):
    b = v_ref.shape[0]
    v = v_ref[...]
    tok = t_ref[...]
    lane = jax.lax.broadcasted_iota(jnp.int32, (b, _BEAM * _BEAM), 1)
    beam = lane // _BEAM
    flat = beam * vocab + tok
    mv, mf = [], []
    for j in range(_BEAM):
        m = jnp.max(v, axis=1, keepdims=True)
        fm = jnp.min(jnp.where(v == m, flat, _BIG), axis=1, keepdims=True)
        mv.append(m)
        mf.append(fm)
        if j < _BEAM - 1:
            v = jnp.where(flat == fm, _NEG, v)
    topv = jnp.concatenate(mv, axis=1)
    topf = jnp.concatenate(mf, axis=1)
    beam_id = topf // vocab
    token = topf - beam_id * vocab
    lp_ref[...] = topv
    sc_ref[...] = topv * _INV_PEN
    tok_ref[...] = token
    rows_ref[...] = beam_id + off_ref[...]
    fin_ref[...] = (token == _END).astype(jnp.int32)
    acc = jnp.zeros((b, _BEAM, cur_len), jnp.int32)
    for j in range(_BEAM):
        sel = (beam_id == j).astype(jnp.int32)
        row = gb_ref[:, j, :]
        acc = acc + sel[:, :, None] * row[:, None, :]
    gb_out_ref[...] = jnp.concatenate([acc, token[:, :, None]], axis=2)


def kernel(log_probabilities, topk_log_probabilities, growing_beam, beam_offset):
    rows, vocab = log_probabilities.shape
    batch = rows // _BEAM
    cur_len = growing_beam.shape[1]
    nchunks = pl.cdiv(vocab, _CHUNK)

    tk = topk_log_probabilities.reshape(rows, 1)
    val, idx, fb = pl.pallas_call(
        lambda *refs: _scan_body(vocab, nchunks, *refs),
        grid=(nchunks,),
        in_specs=[
            pl.BlockSpec((rows, _CHUNK), lambda i: (0, i)),
            pl.BlockSpec((rows, 1), lambda i: (0, 0)),
        ],
        out_specs=[
            pl.BlockSpec((rows, _BEAM), lambda i: (0, 0)),
            pl.BlockSpec((rows, _BEAM), lambda i: (0, 0)),
            pl.BlockSpec((rows, 1), lambda i: (0, 0)),
        ],
        out_shape=[
            jax.ShapeDtypeStruct((rows, _BEAM), jnp.float32),
            jax.ShapeDtypeStruct((rows, _BEAM), jnp.int32),
            jax.ShapeDtypeStruct((rows, 1), jnp.int32),
        ],
        scratch_shapes=[
            pltpu.VMEM((_TREES, 2, rows, _LANES), jnp.float32),
            pltpu.VMEM((_TREES, 2, rows, _LANES), jnp.int32),
            pltpu.VMEM((_TREES, rows, _LANES), jnp.float32),
        ],
    )(log_probabilities, tk)

    def _slow():
        return pl.pallas_call(
            lambda *refs: _slow_body(vocab, *refs),
            grid=(nchunks,),
            in_specs=[
                pl.BlockSpec((rows, _CHUNK), lambda i: (0, i)),
                pl.BlockSpec((rows, 1), lambda i: (0, 0)),
            ],
            out_specs=[
                pl.BlockSpec((rows, _BEAM), lambda i: (0, 0)),
                pl.BlockSpec((rows, _BEAM), lambda i: (0, 0)),
            ],
            out_shape=[
                jax.ShapeDtypeStruct((rows, _BEAM), jnp.float32),
                jax.ShapeDtypeStruct((rows, _BEAM), jnp.int32),
            ],
        )(log_probabilities, tk)

    val, idx = jax.lax.cond(jnp.any(fb != 0), _slow, lambda: (val, idx))

    v16 = val.reshape(batch, _BEAM * _BEAM)
    t16 = idx.reshape(batch, _BEAM * _BEAM)
    gb3 = growing_beam.reshape(batch, _BEAM, cur_len)
    off = beam_offset.reshape(batch, 1)

    topv, scores, token, srows, newgb, fin = pl.pallas_call(
        lambda *refs: _merge_body(vocab, cur_len, *refs),
        out_shape=[
            jax.ShapeDtypeStruct((batch, _BEAM), jnp.float32),
            jax.ShapeDtypeStruct((batch, _BEAM), jnp.float32),
            jax.ShapeDtypeStruct((batch, _BEAM), jnp.int32),
            jax.ShapeDtypeStruct((batch, _BEAM), jnp.int32),
            jax.ShapeDtypeStruct((batch, _BEAM, cur_len + 1), jnp.int32),
            jax.ShapeDtypeStruct((batch, _BEAM), jnp.int32),
        ],
    )(v16, t16, gb3, off)

    return (topv, scores, token,
            srows.reshape(-1),
            newgb.reshape(rows, cur_len + 1),
            fin.astype(bool))
